# baseline (speedup 1.0000x reference)
import jax
import jax.numpy as jnp
from jax.experimental import pallas as pl
from jax.experimental.pallas import tpu as pltpu

_BM = 40
_NBUF = 5


def _gcn_body(a_hbm, e_ref, o_ref, a_buf, sems):
    nblk = a_hbm.shape[0] // _BM
    nrot = nblk // _NBUF

    def copy(slot, idx):
        return pltpu.make_async_copy(
            a_hbm.at[pl.ds(idx * _BM, _BM), :],
            a_buf.at[slot],
            sems.at[slot],
        )

    def stage(slot, idx):
        copy(slot, idx).wait()
        o_ref[pl.ds(idx * _BM, _BM), :] = jnp.dot(
            a_buf[slot], e_ref[...], preferred_element_type=jnp.float32)

    for s in range(_NBUF - 1):
        copy(s, s).start()

    def rotation(i, carry):
        base = i * _NBUF
        for s in range(_NBUF):
            copy((s + _NBUF - 1) % _NBUF, base + s + _NBUF - 1).start()
            stage(s, base + s)
        return carry

    jax.lax.fori_loop(0, nrot - 1, rotation, 0)
    base = (nrot - 1) * _NBUF
    for s in range(_NBUF):
        if s == 0:
            copy(_NBUF - 1, base + _NBUF - 1).start()
        stage(s, base + s)


def kernel(A_hat, E):
    n, k = A_hat.shape
    d = E.shape[1]
    return pl.pallas_call(
        _gcn_body,
        in_specs=[
            pl.BlockSpec(memory_space=pltpu.MemorySpace.HBM),
            pl.BlockSpec(memory_space=pltpu.MemorySpace.VMEM),
        ],
        out_specs=pl.BlockSpec(memory_space=pltpu.MemorySpace.VMEM),
        out_shape=jax.ShapeDtypeStruct((n, d), jnp.float32),
        scratch_shapes=[
            pltpu.MemorySpace.VMEM((_NBUF, _BM, k), jnp.float32),
            pltpu.SemaphoreType.DMA((_NBUF,)),
        ],
    )(A_hat, E)


# speedup vs baseline: 1.0115x; 1.0115x over previous
import jax
import jax.numpy as jnp
from jax.experimental import pallas as pl
from jax.experimental.pallas import tpu as pltpu

_BR = 80
_NR = 5
_BMID = 400
_RAMP = _BR * _NR
_NMID = 23


def _gcn_body(a_hbm, e_ref, o_ref, r_buf, m_buf, r_sems, m_sems):
    def rcopy(slot, row):
        return pltpu.make_async_copy(
            a_hbm.at[pl.ds(row, _BR), :], r_buf.at[slot], r_sems.at[slot])

    def mcopy(slot, row):
        return pltpu.make_async_copy(
            a_hbm.at[pl.ds(row, _BMID), :], m_buf.at[slot], m_sems.at[slot])

    def rdot(slot, row):
        rcopy(slot, row).wait()
        o_ref[pl.ds(row, _BR), :] = jnp.dot(
            r_buf[slot], e_ref[...], preferred_element_type=jnp.float32)

    def mdot(slot, row):
        mcopy(slot, row).wait()
        o_ref[pl.ds(row, _BMID), :] = jnp.dot(
            m_buf[slot], e_ref[...], preferred_element_type=jnp.float32)

    for s in range(_NR):
        rcopy(s, s * _BR).start()
    mcopy(0, _RAMP).start()
    mcopy(1, _RAMP + _BMID).start()
    for s in range(_NR):
        rdot(s, s * _BR)

    def rotation(i, carry):
        base = _RAMP + 2 * i * _BMID
        mdot(0, base)
        mcopy(0, base + 2 * _BMID).start()
        mdot(1, base + _BMID)

        @pl.when(i < (_NMID - 1) // 2 - 1)
        def _():
            mcopy(1, base + 3 * _BMID).start()

        return carry

    jax.lax.fori_loop(0, (_NMID - 1) // 2, rotation, 0)
    tail = a_hbm.shape[0] - _RAMP
    for s in range(_NR):
        rcopy(s, tail + s * _BR).start()
    mdot(0, _RAMP + (_NMID - 1) * _BMID)
    for s in range(_NR):
        rdot(s, tail + s * _BR)


def kernel(A_hat, E):
    n, k = A_hat.shape
    d = E.shape[1]
    return pl.pallas_call(
        _gcn_body,
        in_specs=[
            pl.BlockSpec(memory_space=pltpu.MemorySpace.HBM),
            pl.BlockSpec(memory_space=pltpu.MemorySpace.VMEM),
        ],
        out_specs=pl.BlockSpec(memory_space=pltpu.MemorySpace.VMEM),
        out_shape=jax.ShapeDtypeStruct((n, d), jnp.float32),
        scratch_shapes=[
            pltpu.MemorySpace.VMEM((_NR, _BR, k), jnp.float32),
            pltpu.MemorySpace.VMEM((2, _BMID, k), jnp.float32),
            pltpu.SemaphoreType.DMA((_NR,)),
            pltpu.SemaphoreType.DMA((2,)),
        ],
    )(A_hat, E)


# speedup vs baseline: 1.0781x; 1.0659x over previous
import jax
import jax.numpy as jnp
from jax.experimental import pallas as pl
from jax.experimental.pallas import tpu as pltpu

_BM = 80
_NBUF = 5


def _gcn_body(a_hbm, e_ref, o_ref, a_buf, sems):
    nblk = a_hbm.shape[0] // _BM
    nrot = nblk // _NBUF

    def copy(slot, idx):
        return pltpu.make_async_copy(
            a_hbm.at[pl.ds(idx * _BM, _BM), :],
            a_buf.at[slot],
            sems.at[slot],
        )

    def stage(slot, idx):
        copy(slot, idx).wait()
        o_ref[pl.ds(idx * _BM, _BM), :] = jnp.dot(
            a_buf[slot], e_ref[...], preferred_element_type=jnp.float32)

    for s in range(_NBUF - 1):
        copy(s, s).start()

    def rotation(i, carry):
        base = i * _NBUF
        for s in range(_NBUF):
            copy((s + _NBUF - 1) % _NBUF, base + s + _NBUF - 1).start()
            stage(s, base + s)
        return carry

    jax.lax.fori_loop(0, nrot - 1, rotation, 0)
    base = (nrot - 1) * _NBUF
    for s in range(_NBUF):
        if s == 0:
            copy(_NBUF - 1, base + _NBUF - 1).start()
        stage(s, base + s)


def kernel(A_hat, E):
    n, k = A_hat.shape
    d = E.shape[1]
    return pl.pallas_call(
        _gcn_body,
        in_specs=[
            pl.BlockSpec(memory_space=pltpu.MemorySpace.HBM),
            pl.BlockSpec(memory_space=pltpu.MemorySpace.VMEM),
        ],
        out_specs=pl.BlockSpec(memory_space=pltpu.MemorySpace.VMEM),
        out_shape=jax.ShapeDtypeStruct((n, d), jnp.float32),
        scratch_shapes=[
            pltpu.MemorySpace.VMEM((_NBUF, _BM, k), jnp.float32),
            pltpu.SemaphoreType.DMA((_NBUF,)),
        ],
    )(A_hat, E)


# speedup vs baseline: 1.0803x; 1.0020x over previous
import jax
import jax.numpy as jnp
from jax.experimental import pallas as pl
from jax.experimental.pallas import tpu as pltpu

_BM = 80
_NBUF = 5


def _gcn_body(a_hbm, e_ref, o_ref, a_buf, e16_ref, sems):
    nblk = a_hbm.shape[0] // _BM
    nrot = nblk // _NBUF
    e16_ref[...] = e_ref[...].astype(jnp.bfloat16)

    def copy(slot, idx):
        return pltpu.make_async_copy(
            a_hbm.at[pl.ds(idx * _BM, _BM), :],
            a_buf.at[slot],
            sems.at[slot],
        )

    def stage(slot, idx):
        copy(slot, idx).wait()
        o_ref[pl.ds(idx * _BM, _BM), :] = jnp.dot(
            a_buf[slot].astype(jnp.bfloat16), e16_ref[...],
            preferred_element_type=jnp.float32)

    for s in range(_NBUF - 1):
        copy(s, s).start()

    def rotation(i, carry):
        base = i * _NBUF
        for s in range(_NBUF):
            copy((s + _NBUF - 1) % _NBUF, base + s + _NBUF - 1).start()
            stage(s, base + s)
        return carry

    jax.lax.fori_loop(0, nrot - 1, rotation, 0)
    base = (nrot - 1) * _NBUF
    for s in range(_NBUF):
        if s == 0:
            copy(_NBUF - 1, base + _NBUF - 1).start()
        stage(s, base + s)


def kernel(A_hat, E):
    n, k = A_hat.shape
    d = E.shape[1]
    return pl.pallas_call(
        _gcn_body,
        in_specs=[
            pl.BlockSpec(memory_space=pltpu.MemorySpace.HBM),
            pl.BlockSpec(memory_space=pltpu.MemorySpace.VMEM),
        ],
        out_specs=pl.BlockSpec(memory_space=pltpu.MemorySpace.VMEM),
        out_shape=jax.ShapeDtypeStruct((n, d), jnp.float32),
        scratch_shapes=[
            pltpu.MemorySpace.VMEM((_NBUF, _BM, k), jnp.float32),
            pltpu.MemorySpace.VMEM((k, d), jnp.bfloat16),
            pltpu.SemaphoreType.DMA((_NBUF,)),
        ],
    )(A_hat, E)
